# baseline (speedup 1.0000x reference)
import jax
import jax.numpy as jnp
from jax.experimental import pallas as pl
from jax.experimental.pallas import tpu as pltpu

_N_EVENTS = 4096
_N_NEURONS = 1024
_TS = 256


def _body(all_ts_ref, w_in_ref, ch_in_ref, out_ref, w_ref, ch_ref, wn2_ref):
    w_ref[...] = w_in_ref[...]
    ch_ref[...] = ch_in_ref[...]
    wn2_ref[...] = jnp.sum(w_in_ref[...] * w_in_ref[...], axis=1)
    chsum0 = jnp.sum(ch_in_ref[...])
    iota_n = jax.lax.iota(jnp.int32, _N_NEURONS)

    def step(t, carry):
        ts = all_ts_ref[pl.ds(t, 1), :]
        s = jnp.sqrt(jnp.sum(ts * ts))
        tsd = ts / s
        w = w_ref[...]
        mv = jnp.sum(w * tsd, axis=1)
        beta = mv * jax.lax.rsqrt(wn2_ref[...])
        ch = ch_ref[...]
        chsum = chsum0 + t.astype(jnp.float32)
        gain = jnp.exp((1.0 - (ch * 1024.0) / chsum) * 0.25)
        bh = gain * beta
        n = jnp.argmax(bh).astype(jnp.int32)
        onehot = iota_n == n
        ch_n = jnp.sum(jnp.where(onehot, ch, 0.0))
        beta_n = jnp.sum(jnp.where(onehot, beta, 0.0))
        alpha = jnp.float32(0.01) / (1.0 + ch_n * jnp.float32(5e-5))
        a = alpha * beta_n
        ck = w_ref[pl.ds(n, 1), :]
        newrow = ck + a * (tsd - ck)
        w_ref[pl.ds(n, 1), :] = newrow
        wn2_ref[...] = jnp.where(
            onehot, jnp.sum(newrow * newrow), wn2_ref[...])
        ch_ref[...] = jnp.where(onehot, ch + 1.0, ch)
        out_ref[0] = n
        return carry

    jax.lax.fori_loop(0, _N_EVENTS, step, jnp.int32(0))


def kernel(all_ts, W, cumhisto):
    out = pl.pallas_call(
        _body,
        out_shape=jax.ShapeDtypeStruct((1,), jnp.int32),
        in_specs=[
            pl.BlockSpec(memory_space=pltpu.VMEM),
            pl.BlockSpec(memory_space=pltpu.VMEM),
            pl.BlockSpec(memory_space=pltpu.VMEM),
        ],
        out_specs=pl.BlockSpec(memory_space=pltpu.SMEM),
        scratch_shapes=[
            pltpu.VMEM((_N_NEURONS, _TS), jnp.float32),
            pltpu.VMEM((_N_NEURONS,), jnp.float32),
            pltpu.VMEM((_N_NEURONS,), jnp.float32),
        ],
    )(all_ts, W, cumhisto)
    return out[0]


# speedup vs baseline: 1.2306x; 1.2306x over previous
import jax
import jax.numpy as jnp
from jax.experimental import pallas as pl
from jax.experimental.pallas import tpu as pltpu

_N_EVENTS = 4096
_N_NEURONS = 1024
_TS = 256


def _body(all_ts_ref, w_in_ref, ch_in_ref, out_ref,
          w_ref, wt_ref, ch_ref, wn2_ref):
    w_ref[...] = w_in_ref[...]
    wt_ref[...] = w_in_ref[...].T
    ch_ref[...] = ch_in_ref[...]
    wn2_ref[...] = jnp.sum(w_in_ref[...] * w_in_ref[...], axis=1)
    chsum0 = jnp.sum(ch_in_ref[...])
    iota_n = jax.lax.iota(jnp.int32, _N_NEURONS)
    lane_iota = jax.lax.broadcasted_iota(jnp.int32, (_TS, 128), 1)

    def _pick(sel, a, b):
        return jnp.where(sel, a, b)

    def step(t, carry):
        ts = all_ts_ref[pl.ds(t, 1), :]
        s = jnp.sqrt(jnp.sum(ts * ts))
        tsd = ts / s
        tsd_t = tsd.reshape(_TS, 1)
        mv_fast = jnp.sum(wt_ref[...] * tsd_t, axis=0)
        rinv = jax.lax.rsqrt(wn2_ref[...])
        ch = ch_ref[...]
        chsum = chsum0 + t.astype(jnp.float32)
        gain = jnp.exp((1.0 - (ch * 1024.0) / chsum) * 0.25)
        bh_fast = gain * (mv_fast * rinv)
        n1 = jnp.argmax(bh_fast).astype(jnp.int32)
        m1 = iota_n == n1
        n2 = jnp.argmax(jnp.where(m1, -jnp.inf, bh_fast)).astype(jnp.int32)
        m2 = iota_n == n2
        row1 = w_ref[pl.ds(n1, 1), :]
        row2 = w_ref[pl.ds(n2, 1), :]
        b1 = jnp.sum(row1 * tsd) * jnp.sum(jnp.where(m1, rinv, 0.0))
        b2 = jnp.sum(row2 * tsd) * jnp.sum(jnp.where(m2, rinv, 0.0))
        bh1 = jnp.sum(jnp.where(m1, gain, 0.0)) * b1
        bh2 = jnp.sum(jnp.where(m2, gain, 0.0)) * b2
        lo1 = n1 < n2
        n_lo = _pick(lo1, n1, n2)
        n_hi = _pick(lo1, n2, n1)
        bh_lo = _pick(lo1, bh1, bh2)
        bh_hi = _pick(lo1, bh2, bh1)
        win_lo = bh_lo >= bh_hi
        n = _pick(win_lo, n_lo, n_hi)
        beta_n = _pick(win_lo, _pick(lo1, b1, b2), _pick(lo1, b2, b1))
        onehot = iota_n == n
        ch_n = jnp.sum(jnp.where(onehot, ch, 0.0))
        alpha = jnp.float32(0.01) / (1.0 + ch_n * jnp.float32(5e-5))
        a = alpha * beta_n
        ck = _pick(n == n1, row1, row2)
        newrow = ck + a * (tsd - ck)
        w_ref[pl.ds(n, 1), :] = newrow
        tile = (n // 128) * 128
        newrow_t = newrow.reshape(_TS, 1)
        wt_ref[:, pl.ds(tile, 128)] = jnp.where(
            lane_iota == (n % 128), newrow_t, wt_ref[:, pl.ds(tile, 128)])
        wn2_ref[...] = jnp.where(
            onehot, jnp.sum(newrow * newrow), wn2_ref[...])
        ch_ref[...] = jnp.where(onehot, ch + 1.0, ch)
        out_ref[0] = n
        return carry

    jax.lax.fori_loop(0, _N_EVENTS, step, jnp.int32(0))


def kernel(all_ts, W, cumhisto):
    out = pl.pallas_call(
        _body,
        out_shape=jax.ShapeDtypeStruct((1,), jnp.int32),
        in_specs=[
            pl.BlockSpec(memory_space=pltpu.VMEM),
            pl.BlockSpec(memory_space=pltpu.VMEM),
            pl.BlockSpec(memory_space=pltpu.VMEM),
        ],
        out_specs=pl.BlockSpec(memory_space=pltpu.SMEM),
        scratch_shapes=[
            pltpu.VMEM((_N_NEURONS, _TS), jnp.float32),
            pltpu.VMEM((_TS, _N_NEURONS), jnp.float32),
            pltpu.VMEM((_N_NEURONS,), jnp.float32),
            pltpu.VMEM((_N_NEURONS,), jnp.float32),
        ],
    )(all_ts, W, cumhisto)
    return out[0]
